# baseline (speedup 1.0000x reference)
import functools

import jax
import jax.numpy as jnp
from jax import lax
from jax.experimental import pallas as pl
from jax.experimental.pallas import tpu as pltpu
from jax.experimental.pallas import tpu_sc as plsc

D = 64


def _make_lookup(b: int, s: int):
    info = plsc.get_sparse_core_info()
    nc, ns = info.num_cores, info.num_subcores
    nw = nc * ns
    assert b % nw == 0
    rows_per_w = b // nw

    mesh = plsc.VectorSubcoreMesh(core_axis_name="c", subcore_axis_name="s")

    nb = 8
    assert rows_per_w % nb == 0
    nrounds = rows_per_w // nb

    @functools.partial(
        pl.kernel,
        mesh=mesh,
        compiler_params=pltpu.CompilerParams(use_tc_tiling_on_sc=False),
        out_type=jax.ShapeDtypeStruct((b, s, D), jnp.float32),
        scratch_types=[
            pltpu.VMEM((rows_per_w, s), jnp.int32),
            pltpu.VMEM((nb, s, D), jnp.float32),
            pltpu.SemaphoreType.DMA((nb,)),
            pltpu.SemaphoreType.DMA((nb,)),
        ],
    )
    def lookup(tok_hbm, table_hbm, out_hbm, idx_v, rows_v, gsem, ssem):
        wid = lax.axis_index("s") * nc + lax.axis_index("c")
        base_row = wid * rows_per_w
        pltpu.sync_copy(tok_hbm.at[pl.ds(base_row, rows_per_w)], idx_v)

        def gather(r, bf):
            return pltpu.async_copy(table_hbm.at[idx_v.at[r]], rows_v.at[bf],
                                    gsem.at[bf])

        def scatter(r, bf):
            return pltpu.async_copy(rows_v.at[bf], out_hbm.at[base_row + r],
                                    ssem.at[bf])

        def wait_gather(r, bf):
            pltpu.make_async_copy(table_hbm.at[idx_v.at[r]], rows_v.at[bf],
                                  gsem.at[bf]).wait()

        def wait_scatter(r, bf):
            pltpu.make_async_copy(rows_v.at[bf], out_hbm.at[base_row + r],
                                  ssem.at[bf]).wait()

        for bf in range(nb):
            gather(bf, bf)

        def round_fn(g, carry):
            r0 = g * nb
            for bf in range(nb):
                wait_gather(r0 + bf, bf)
                scatter(r0 + bf, bf)
            for bf in range(nb):
                wait_scatter(r0 + bf, bf)

                @pl.when(g + 1 < nrounds)
                def _():
                    gather(r0 + nb + bf, bf)
            return carry

        lax.fori_loop(0, nrounds, round_fn, 0)

    return lookup


def kernel(token_ids, weight):
    b, s = token_ids.shape
    lookup = _make_lookup(b, s)
    return lookup(token_ids, weight)


# speedup vs baseline: 1.1036x; 1.1036x over previous
import functools

import jax
import jax.numpy as jnp
from jax import lax
from jax.experimental import pallas as pl
from jax.experimental.pallas import tpu as pltpu
from jax.experimental.pallas import tpu_sc as plsc

D = 64
WD = 128
L = 16


def _make_lookup(b: int, s: int, v: int):
    info = plsc.get_sparse_core_info()
    nc, ns = info.num_cores, info.num_subcores
    nw = nc * ns
    assert b % nw == 0
    rows_per_w = b // nw

    mesh = plsc.VectorSubcoreMesh(core_axis_name="c", subcore_axis_name="s")

    nb = 4
    assert rows_per_w % nb == 0
    nrounds = rows_per_w // nb

    @functools.partial(
        pl.kernel,
        mesh=mesh,
        out_type=jax.ShapeDtypeStruct((b, s, D), jnp.float32),
        scratch_types=[
            pltpu.VMEM((rows_per_w, s), jnp.int32),
            pltpu.VMEM((nb, s, WD), jnp.float32),
            pltpu.VMEM((nb, s, D), jnp.float32),
            pltpu.SemaphoreType.DMA((nb,)),
            pltpu.SemaphoreType.DMA((nb,)),
        ],
    )
    def lookup(tok_hbm, wide_hbm, out_hbm, idx_v, rows_v, pack_v, gsem, ssem):
        wid = lax.axis_index("s") * nc + lax.axis_index("c")
        base_row = wid * rows_per_w
        pltpu.sync_copy(tok_hbm.at[pl.ds(base_row, rows_per_w)], idx_v)

        def gather(r, bf):
            return pltpu.async_copy(wide_hbm.at[idx_v.at[r]], rows_v.at[bf],
                                    gsem.at[bf])

        def scatter(r, bf):
            return pltpu.async_copy(pack_v.at[bf], out_hbm.at[base_row + r],
                                    ssem.at[bf])

        def wait_gather(r, bf):
            pltpu.make_async_copy(wide_hbm.at[idx_v.at[r]], rows_v.at[bf],
                                  gsem.at[bf]).wait()

        def wait_scatter(r, bf):
            pltpu.make_async_copy(pack_v.at[bf], out_hbm.at[base_row + r],
                                  ssem.at[bf]).wait()

        def pack(bf):
            def row_pair(i, carry):
                for ii in range(2):
                    for c in range(D // L):
                        pack_v[bf, i * 2 + ii, pl.ds(c * L, L)] = (
                            rows_v[bf, i * 2 + ii, pl.ds(c * L, L)])
                return carry

            lax.fori_loop(0, s // 2, row_pair, 0)
            if s % 2:
                for c in range(D // L):
                    pack_v[bf, s - 1, pl.ds(c * L, L)] = (
                        rows_v[bf, s - 1, pl.ds(c * L, L)])

        for bf in range(nb):
            gather(bf, bf)

        def round_fn(g, carry):
            r0 = g * nb
            for bf in range(nb):
                wait_gather(r0 + bf, bf)

                @pl.when(g > 0)
                def _():
                    wait_scatter(r0 - nb + bf, bf)

                pack(bf)
                scatter(r0 + bf, bf)

                @pl.when(g + 1 < nrounds)
                def _():
                    gather(r0 + nb + bf, bf)
            return carry

        lax.fori_loop(0, nrounds, round_fn, 0)

        for bf in range(nb):
            wait_scatter((nrounds - 1) * nb + bf, bf)

    return lookup


def kernel(token_ids, weight):
    b, s = token_ids.shape
    v, d = weight.shape
    assert d == D
    wide = jnp.pad(weight, ((0, 0), (0, WD - D)))
    return _make_lookup(b, s, v)(token_ids, wide)


# speedup vs baseline: 1.1050x; 1.0012x over previous
import functools

import jax
import jax.numpy as jnp
from jax import lax
from jax.experimental import pallas as pl
from jax.experimental.pallas import tpu as pltpu
from jax.experimental.pallas import tpu_sc as plsc

D = 64
WD = 128
L = 16


def _make_lookup(b: int, s: int, v: int):
    info = plsc.get_sparse_core_info()
    nc, ns = info.num_cores, info.num_subcores
    nw = nc * ns
    assert b % nw == 0
    rows_per_w = b // nw

    mesh = plsc.VectorSubcoreMesh(core_axis_name="c", subcore_axis_name="s")

    nb = 4
    assert rows_per_w % nb == 0
    nrounds = rows_per_w // nb

    @functools.partial(
        pl.kernel,
        mesh=mesh,
        out_type=jax.ShapeDtypeStruct((b, s, D), jnp.float32),
        scratch_types=[
            pltpu.VMEM((rows_per_w, s), jnp.int32),
            pltpu.VMEM((nb, s, WD), jnp.float32),
            pltpu.VMEM((nb, s, D), jnp.float32),
            pltpu.SemaphoreType.DMA((nb,)),
            pltpu.SemaphoreType.DMA((nb,)),
        ],
    )
    def lookup(tok_hbm, wide_hbm, out_hbm, idx_v, rows_v, pack_v, gsem, ssem):
        wid = lax.axis_index("s") * nc + lax.axis_index("c")
        base_row = wid * rows_per_w
        pltpu.sync_copy(tok_hbm.at[pl.ds(base_row, rows_per_w)], idx_v)

        def gather(r, bf):
            return pltpu.async_copy(wide_hbm.at[idx_v.at[r]], rows_v.at[bf],
                                    gsem.at[bf])

        def scatter(r, bf):
            return pltpu.async_copy(pack_v.at[bf], out_hbm.at[base_row + r],
                                    ssem.at[bf])

        def wait_gather(r, bf):
            pltpu.make_async_copy(wide_hbm.at[idx_v.at[r]], rows_v.at[bf],
                                  gsem.at[bf]).wait()

        def wait_scatter(r, bf):
            pltpu.make_async_copy(pack_v.at[bf], out_hbm.at[base_row + r],
                                  ssem.at[bf]).wait()

        def pack(bf):
            def row_pair(i, carry):
                for ii in range(2):
                    for c in range(D // L):
                        pack_v[bf, i * 2 + ii, pl.ds(c * L, L)] = (
                            rows_v[bf, i * 2 + ii, pl.ds(c * L, L)])
                return carry

            lax.fori_loop(0, s // 2, row_pair, 0)
            if s % 2:
                for c in range(D // L):
                    pack_v[bf, s - 1, pl.ds(c * L, L)] = (
                        rows_v[bf, s - 1, pl.ds(c * L, L)])

        for bf in range(nb):
            gather(bf, bf)

        def round_fn(g, carry):
            r0 = g * nb
            for bf in range(nb):
                wait_gather(r0 + bf, bf)

                @pl.when(g > 0)
                def _():
                    wait_scatter(r0 - nb + bf, bf)

                pack(bf)
                scatter(r0 + bf, bf)

                @pl.when(g + 1 < nrounds)
                def _():
                    gather(r0 + nb + bf, bf)
            return carry

        lax.fori_loop(0, nrounds, round_fn, 0)

        for bf in range(nb):
            wait_scatter((nrounds - 1) * nb + bf, bf)

    return lookup


def kernel(token_ids, weight):
    b, s = token_ids.shape
    v, d = weight.shape
    assert d == D
    wide = lax.optimization_barrier(jnp.pad(weight, ((0, 0), (0, WD - D))))
    return _make_lookup(b, s, v)(token_ids, wide)


# speedup vs baseline: 1.4548x; 1.3166x over previous
import functools

import jax
import jax.numpy as jnp
from jax import lax
from jax.experimental import pallas as pl
from jax.experimental.pallas import tpu as pltpu
from jax.experimental.pallas import tpu_sc as plsc

D = 64
WD = 128
L = 16


def _make_transpose_widen(v: int):
    tblk = 16000

    def body(wt_ref, out_ref):
        x = wt_ref[...]
        xt = x.T
        out_ref[:, 0:D] = xt
        out_ref[:, D:WD] = jnp.zeros((tblk, WD - D), jnp.float32)

    return pl.pallas_call(
        body,
        grid=((v + tblk - 1) // tblk,),
        in_specs=[pl.BlockSpec((D, tblk), lambda i: (0, i))],
        out_specs=pl.BlockSpec((tblk, WD), lambda i: (i, 0)),
        out_shape=jax.ShapeDtypeStruct((v, WD), jnp.float32),
    )


def _make_lookup(b: int, s: int, v: int):
    info = plsc.get_sparse_core_info()
    nc, ns = info.num_cores, info.num_subcores
    nw = nc * ns
    assert b % nw == 0
    rows_per_w = b // nw

    mesh = plsc.VectorSubcoreMesh(core_axis_name="c", subcore_axis_name="s")

    nb = 4
    assert rows_per_w % nb == 0
    nrounds = rows_per_w // nb

    @functools.partial(
        pl.kernel,
        mesh=mesh,
        out_type=jax.ShapeDtypeStruct((b, s, D), jnp.float32),
        scratch_types=[
            pltpu.VMEM((rows_per_w, s), jnp.int32),
            pltpu.VMEM((nb, s, WD), jnp.float32),
            pltpu.VMEM((nb, s, D), jnp.float32),
            pltpu.SemaphoreType.DMA((nb,)),
            pltpu.SemaphoreType.DMA((nb,)),
        ],
    )
    def lookup(tok_hbm, wide_hbm, out_hbm, idx_v, rows_v, pack_v, gsem, ssem):
        wid = lax.axis_index("s") * nc + lax.axis_index("c")
        base_row = wid * rows_per_w
        pltpu.sync_copy(tok_hbm.at[pl.ds(base_row, rows_per_w)], idx_v)

        def gather(r, bf):
            return pltpu.async_copy(wide_hbm.at[idx_v.at[r]], rows_v.at[bf],
                                    gsem.at[bf])

        def scatter(r, bf):
            return pltpu.async_copy(pack_v.at[bf], out_hbm.at[base_row + r],
                                    ssem.at[bf])

        def wait_gather(r, bf):
            pltpu.make_async_copy(wide_hbm.at[idx_v.at[r]], rows_v.at[bf],
                                  gsem.at[bf]).wait()

        def wait_scatter(r, bf):
            pltpu.make_async_copy(pack_v.at[bf], out_hbm.at[base_row + r],
                                  ssem.at[bf]).wait()

        def pack(bf):
            def row_pair(i, carry):
                for ii in range(2):
                    for c in range(D // L):
                        pack_v[bf, i * 2 + ii, pl.ds(c * L, L)] = (
                            rows_v[bf, i * 2 + ii, pl.ds(c * L, L)])
                return carry

            lax.fori_loop(0, s // 2, row_pair, 0)
            if s % 2:
                for c in range(D // L):
                    pack_v[bf, s - 1, pl.ds(c * L, L)] = (
                        rows_v[bf, s - 1, pl.ds(c * L, L)])

        for bf in range(nb):
            gather(bf, bf)

        def round_fn(g, carry):
            r0 = g * nb
            for bf in range(nb):
                wait_gather(r0 + bf, bf)

                @pl.when(g > 0)
                def _():
                    wait_scatter(r0 - nb + bf, bf)

                pack(bf)
                scatter(r0 + bf, bf)

                @pl.when(g + 1 < nrounds)
                def _():
                    gather(r0 + nb + bf, bf)
            return carry

        lax.fori_loop(0, nrounds, round_fn, 0)

        for bf in range(nb):
            wait_scatter((nrounds - 1) * nb + bf, bf)

    return lookup


def kernel(token_ids, weight):
    b, s = token_ids.shape
    v, d = weight.shape
    assert d == D
    wide = _make_transpose_widen(v)(weight.T)
    return _make_lookup(b, s, v)(token_ids, wide)
